# baseline (speedup 1.0000x reference)
import jax
import jax.numpy as jnp
import numpy as np
from jax.experimental import pallas as pl
from jax.experimental.pallas import tpu as pltpu

_LAYER_CFG = [
    (1, 32, 8, 1, 256),
    (32, 64, 8, 2, 256),
    (64, 96, 8, 4, 256),
    (96, 128, 12, 4, 120),
    (128, 160, 12, 6, 120),
]

_SEL_CONST = {
    (li, n): np.asarray(jax.random.permutation(
        jax.random.fold_in(jax.random.key(1), li), n))
    for li, n in ((0, 1024), (3, 256))
}


def _sel_const(li, n):
    return _SEL_CONST[(li, n)]


def _fc_head_kernel(fts_ref, w1_ref, b1_ref, w2_ref, b2_ref, w3_ref, b3_ref,
                    out_ref, *, B, P):
    f = fts_ref[...]
    f = jnp.maximum(jnp.dot(f, w1_ref[...],
                            preferred_element_type=jnp.float32) + b1_ref[...], 0.0)
    f = jnp.maximum(jnp.dot(f, w2_ref[...],
                            preferred_element_type=jnp.float32) + b2_ref[...], 0.0)
    logits = jnp.dot(f, w3_ref[...],
                     preferred_element_type=jnp.float32) + b3_ref[...]
    logits = jnp.mean(logits.reshape(B, P, -1), axis=1)
    m = jnp.max(logits, axis=-1, keepdims=True)
    s = jnp.log(jnp.sum(jnp.exp(logits - m), axis=-1, keepdims=True))
    out_ref[...] = logits - m - s


def _fc_head(fts, params):
    B, P, C = fts.shape
    out = pl.pallas_call(
        lambda *a: _fc_head_kernel(*a, B=B, P=P),
        out_shape=jax.ShapeDtypeStruct((B, 10), jnp.float32),
    )(fts.reshape(B * P, C),
      params['fc1_W'], params['fc1_b'].reshape(1, -1),
      params['fc2_W'], params['fc2_b'].reshape(1, -1),
      params['fc3_W'], params['fc3_b'].reshape(1, -1))
    return out


def _knn_idx(rep, pts, K, D):
    d2 = jnp.sum((rep[:, :, None, :] - pts[:, None, :, :]) ** 2, axis=-1)
    _, idx = jax.lax.top_k(-d2, D * K + 1)
    return idx[:, :, 1::D]


def _gather(x, idx):
    return jax.vmap(lambda a, i: a[i])(x, idx)


def _layer(li, params, pts, fts, C_in, C_out, K, D, P):
    g = lambda n: params['l%d_%s' % (li, n)]
    B, Np, _ = pts.shape
    if 0 < P < Np:
        sel = _sel_const(li, Np)[:P]
        rep = pts[:, sel, :]
    else:
        rep = pts
        P = Np
    fts_d = jax.nn.elu(fts @ g('dense_W') + g('dense_b'))
    idx = _knn_idx(rep, pts, K, D)
    pts_r = _gather(pts, idx)
    fts_r = _gather(fts_d, idx)
    local = pts_r - rep[:, :, None, :]
    l1 = jax.nn.elu(local @ g('lift1_W') + g('lift1_b'))
    l2 = jax.nn.elu(l1 @ g('lift2_W') + g('lift2_b'))
    fts_cat = jnp.concatenate([l2, fts_r], axis=-1)
    X = jax.nn.elu(jnp.einsum('bpkd,dkj->bpj', local, g('xconv_W')) + g('xconv_b'))
    X = jax.nn.elu(X @ g('xd1_W') + g('xd1_b'))
    X = X @ g('xd2_W') + g('xd2_b')
    X = X.reshape(B, P, K, K)
    fts_X = jnp.einsum('bpkl,bplc->bpkc', X, fts_cat)
    dw_W = g('dw_W')
    Cc, dm, _ = dw_W.shape
    dw = jnp.einsum('bpkc,cmk->bpcm', fts_X, dw_W).reshape(B, P, Cc * dm) + g('dw_b')
    out = jax.nn.elu(dw @ g('pw_W'))
    return rep, out


def kernel(pts, fts, params):
    cur_pts, cur_fts = pts, fts
    for li, (C_in, C_out, K, D, P) in enumerate(_LAYER_CFG):
        cur_pts, cur_fts = _layer(li, params, cur_pts, cur_fts, C_in, C_out, K, D, P)
    return _fc_head(cur_fts, params)


# speedup vs baseline: 1.0149x; 1.0149x over previous
import jax
import jax.numpy as jnp
import numpy as np
from jax.experimental import pallas as pl
from jax.experimental.pallas import tpu as pltpu

_LAYER_CFG = [
    (1, 32, 8, 1, 256),
    (32, 64, 8, 2, 256),
    (64, 96, 8, 4, 256),
    (96, 128, 12, 4, 120),
    (128, 160, 12, 6, 120),
]

_SEL_CONST = {
    (li, n): np.asarray(jax.random.permutation(
        jax.random.fold_in(jax.random.key(1), li), n))
    for li, n in ((0, 1024), (3, 256))
}


def _sel_const(li, n):
    return _SEL_CONST[(li, n)]


def _fc_head_kernel(fts_ref, w1_ref, b1_ref, w2_ref, b2_ref, w3_ref, b3_ref,
                    out_ref, *, B, P):
    f = fts_ref[...]
    f = jnp.maximum(jnp.dot(f, w1_ref[...],
                            preferred_element_type=jnp.float32) + b1_ref[...], 0.0)
    f = jnp.maximum(jnp.dot(f, w2_ref[...],
                            preferred_element_type=jnp.float32) + b2_ref[...], 0.0)
    logits = jnp.dot(f, w3_ref[...],
                     preferred_element_type=jnp.float32) + b3_ref[...]
    logits = jnp.mean(logits.reshape(B, P, -1), axis=1)
    m = jnp.max(logits, axis=-1, keepdims=True)
    s = jnp.log(jnp.sum(jnp.exp(logits - m), axis=-1, keepdims=True))
    out_ref[...] = logits - m - s


def _fc_head(fts, params):
    B, P, C = fts.shape
    out = pl.pallas_call(
        lambda *a: _fc_head_kernel(*a, B=B, P=P),
        out_shape=jax.ShapeDtypeStruct((B, 10), jnp.float32),
    )(fts.reshape(B * P, C),
      params['fc1_W'], params['fc1_b'].reshape(1, -1),
      params['fc2_W'], params['fc2_b'].reshape(1, -1),
      params['fc3_W'], params['fc3_b'].reshape(1, -1))
    return out


def _pad_to(x, axis, size, value):
    pad = size - x.shape[axis]
    if pad <= 0:
        return x
    widths = [(0, 0)] * x.ndim
    widths[axis] = (0, pad)
    return jnp.pad(x, widths, constant_values=value)


def _knn_body(rep_ref, pts_ref, out_ref, *, P, Np, K, D):
    Pp = rep_ref.shape[1]
    Npp = pts_ref.shape[2]
    rx = rep_ref[0, :, 0:1]
    ry = rep_ref[0, :, 1:2]
    rz = rep_ref[0, :, 2:3]
    px = pts_ref[0, 0:1, :]
    py = pts_ref[0, 1:2, :]
    pz = pts_ref[0, 2:3, :]
    dx = rx - px
    dy = ry - py
    dz = rz - pz
    d2 = dx * dx + dy * dy + dz * dz
    iota_n = jax.lax.broadcasted_iota(jnp.int32, (Pp, Npp), 1)
    lane_k = jax.lax.broadcasted_iota(jnp.int32, (Pp, out_ref.shape[2]), 1)
    acc = jnp.zeros((Pp, out_ref.shape[2]), jnp.int32)
    inf = jnp.float32(jnp.inf)
    for k in range(D * K + 1):
        m = jnp.min(d2, axis=1, keepdims=True)
        cand = jnp.where(d2 == m, iota_n, jnp.int32(Npp))
        am = jnp.min(cand, axis=1, keepdims=True)
        if k >= 1 and (k - 1) % D == 0:
            j = (k - 1) // D
            acc = jnp.where(lane_k == j, am, acc)
        if k < D * K:
            d2 = jnp.where(iota_n == am, inf, d2)
    out_ref[0] = acc


def _knn_idx_tc(rep, pts, K, D):
    B, P, _ = rep.shape
    Np = pts.shape[1]
    Pp = max(128, ((P + 7) // 8) * 8)
    Npp = max(128, ((Np + 127) // 128) * 128)
    Kp = 16
    repp = _pad_to(rep, 1, Pp, 0.0)
    ptsp = _pad_to(jnp.swapaxes(pts, 1, 2), 2, Npp, 1e30)
    import functools as _ft
    out = pl.pallas_call(
        _ft.partial(_knn_body, P=P, Np=Np, K=K, D=D),
        grid=(B,),
        in_specs=[
            pl.BlockSpec((1, Pp, 3), lambda b: (b, 0, 0)),
            pl.BlockSpec((1, 3, Npp), lambda b: (b, 0, 0)),
        ],
        out_specs=pl.BlockSpec((1, Pp, Kp), lambda b: (b, 0, 0)),
        out_shape=jax.ShapeDtypeStruct((B, Pp, Kp), jnp.int32),
    )(repp, ptsp)
    return out[:, :P, :K]


def _gather(x, idx):
    return jax.vmap(lambda a, i: a[i])(x, idx)


def _layer(li, params, pts, fts, C_in, C_out, K, D, P):
    g = lambda n: params['l%d_%s' % (li, n)]
    B, Np, _ = pts.shape
    if 0 < P < Np:
        sel = _sel_const(li, Np)[:P]
        rep = pts[:, sel, :]
    else:
        rep = pts
        P = Np
    fts_d = jax.nn.elu(fts @ g('dense_W') + g('dense_b'))
    idx = _knn_idx_tc(rep, pts, K, D)
    pts_r = _gather(pts, idx)
    fts_r = _gather(fts_d, idx)
    local = pts_r - rep[:, :, None, :]
    l1 = jax.nn.elu(local @ g('lift1_W') + g('lift1_b'))
    l2 = jax.nn.elu(l1 @ g('lift2_W') + g('lift2_b'))
    fts_cat = jnp.concatenate([l2, fts_r], axis=-1)
    X = jax.nn.elu(jnp.einsum('bpkd,dkj->bpj', local, g('xconv_W')) + g('xconv_b'))
    X = jax.nn.elu(X @ g('xd1_W') + g('xd1_b'))
    X = X @ g('xd2_W') + g('xd2_b')
    X = X.reshape(B, P, K, K)
    fts_X = jnp.einsum('bpkl,bplc->bpkc', X, fts_cat)
    dw_W = g('dw_W')
    Cc, dm, _ = dw_W.shape
    dw = jnp.einsum('bpkc,cmk->bpcm', fts_X, dw_W).reshape(B, P, Cc * dm) + g('dw_b')
    out = jax.nn.elu(dw @ g('pw_W'))
    return rep, out


def kernel(pts, fts, params):
    cur_pts, cur_fts = pts, fts
    for li, (C_in, C_out, K, D, P) in enumerate(_LAYER_CFG):
        cur_pts, cur_fts = _layer(li, params, cur_pts, cur_fts, C_in, C_out, K, D, P)
    return _fc_head(cur_fts, params)
